# Initial kernel scaffold; baseline (speedup 1.0000x reference)
import jax
import jax.numpy as jnp
from jax.experimental import pallas as pl


def kernel(pred_loc, pred_label, gt_loc, gt_label):
    raise NotImplementedError("write your pallas kernel here")


# speedup vs baseline: 2.8817x; 2.8817x over previous
import functools

import jax
import jax.numpy as jnp
from jax import lax
from jax.experimental import pallas as pl
from jax.experimental.pallas import tpu as pltpu

_B, _A, _C = 32, 20000, 21
_NEG_RATIO = 3
_INF_PAT = 0x7F800000


def _ce_loc_kernel(xt_ref, lbl_ref, ploc_ref, gloc_ref, gl4_ref,
                   ce_ref, loc_ref):
    x = xt_ref[0]
    lbl = lbl_ref[0]
    m = jnp.max(x, axis=0, keepdims=True)
    s = jnp.sum(jnp.exp(x - m), axis=0, keepdims=True)
    cls_iota = lax.broadcasted_iota(jnp.int32, (_C, _A), 0)
    picked = jnp.sum(jnp.where(cls_iota == lbl, x, 0.0),
                     axis=0, keepdims=True)
    ce_ref[0] = jnp.log(s) + m - picked

    d = jnp.abs(ploc_ref[0] - gloc_ref[0])
    sl1 = jnp.where(d < 1.0, 0.5 * d * d, d - 0.5)
    pos4 = (gl4_ref[0] > 0).astype(jnp.float32)
    loc_ref[...] = jnp.sum(sl1 * pos4).reshape(1, 1, 1)


def _select_kernel(ce_ref, gl_ref, cls_ref, npos_ref):
    ce = ce_ref[0]
    lbl = gl_ref[0]
    pos = lbl > 0
    npos = jnp.sum(pos.astype(jnp.int32)).reshape(1, 1)
    k = jnp.minimum(_NEG_RATIO * npos, _A - 1)

    pt = jnp.exp(-ce)
    one_m_pt = 1.0 - pt
    focal = one_m_pt * one_m_pt * ce
    losses = jnp.where(pos, 0.0, focal)
    lv = lax.bitcast_convert_type(losses, jnp.int32)

    def body(_, carry):
        lo, hi = carry
        mid = lo + (hi - lo) // 2
        cnt = jnp.sum((lv > mid).astype(jnp.int32)).reshape(1, 1)
        ge = cnt >= k
        return jnp.where(ge, mid, lo), jnp.where(ge, hi, mid)

    lo0 = jnp.full((1, 1), -1, jnp.int32)
    hi0 = jnp.full((1, 1), _INF_PAT, jnp.int32)
    _, t_pat = lax.fori_loop(0, 31, body, (lo0, hi0))

    gt_t = lv > t_pat
    cnt_gt = jnp.sum(gt_t.astype(jnp.int32)).reshape(1, 1)
    remaining = (k - cnt_gt).astype(jnp.float32)

    eq = (lv == t_pat).astype(jnp.float32)
    lane_iota = lax.broadcasted_iota(jnp.int32, (8, 2500), 1)
    scan = eq
    for sh in (1, 2, 4, 8, 16, 32, 64, 128, 256, 512, 1024, 2048):
        rolled = pltpu.roll(scan, sh, axis=1)
        scan = scan + jnp.where(lane_iota >= sh, rolled, 0.0)
    row_tot = jnp.sum(eq, axis=1, keepdims=True)
    sub_iota_r = lax.broadcasted_iota(jnp.int32, (8, 8), 0)
    sub_iota_c = lax.broadcasted_iota(jnp.int32, (8, 8), 1)
    tri = (sub_iota_r > sub_iota_c).astype(jnp.float32)
    row_off = lax.dot_general(tri, row_tot, (((1,), (0,)), ((), ())),
                              preferred_element_type=jnp.float32)
    c_incl = scan + row_off
    tie_sel = jnp.logical_and(eq > 0.0, c_incl <= remaining)

    mask = jnp.logical_or(pos, jnp.logical_or(gt_t, tie_sel))
    cls_sum = jnp.sum(ce * mask.astype(jnp.float32))
    cls_ref[...] = cls_sum.reshape(1, 1, 1)
    npos_ref[...] = npos.astype(jnp.float32).reshape(1, 1, 1)


@jax.jit
def kernel(pred_loc, pred_label, gt_loc, gt_label):
    B, A, C = _B, _A, _C
    xt = jnp.swapaxes(pred_label, 1, 2)
    lbl_row = gt_label.reshape(B, 1, A)
    ploc = pred_loc.reshape(B, 8, A * 4 // 8)
    gloc = gt_loc.reshape(B, 8, A * 4 // 8)
    gl4 = jnp.repeat(gt_label, 4, axis=-1).reshape(B, 8, A * 4 // 8)

    ce, loc_part = pl.pallas_call(
        _ce_loc_kernel,
        grid=(B,),
        in_specs=[
            pl.BlockSpec((1, C, A), lambda b: (b, 0, 0)),
            pl.BlockSpec((1, 1, A), lambda b: (b, 0, 0)),
            pl.BlockSpec((1, 8, A * 4 // 8), lambda b: (b, 0, 0)),
            pl.BlockSpec((1, 8, A * 4 // 8), lambda b: (b, 0, 0)),
            pl.BlockSpec((1, 8, A * 4 // 8), lambda b: (b, 0, 0)),
        ],
        out_specs=[
            pl.BlockSpec((1, 1, A), lambda b: (b, 0, 0)),
            pl.BlockSpec((1, 1, 1), lambda b: (b, 0, 0)),
        ],
        out_shape=[
            jax.ShapeDtypeStruct((B, 1, A), jnp.float32),
            jax.ShapeDtypeStruct((B, 1, 1), jnp.float32),
        ],
    )(xt, lbl_row, ploc, gloc, gl4)

    ce_r = ce.reshape(B, 8, A // 8)
    gl_r = gt_label.reshape(B, 8, A // 8)
    cls_part, npos = pl.pallas_call(
        _select_kernel,
        grid=(B,),
        in_specs=[
            pl.BlockSpec((1, 8, A // 8), lambda b: (b, 0, 0)),
            pl.BlockSpec((1, 8, A // 8), lambda b: (b, 0, 0)),
        ],
        out_specs=[
            pl.BlockSpec((1, 1, 1), lambda b: (b, 0, 0)),
            pl.BlockSpec((1, 1, 1), lambda b: (b, 0, 0)),
        ],
        out_shape=[
            jax.ShapeDtypeStruct((B, 1, 1), jnp.float32),
            jax.ShapeDtypeStruct((B, 1, 1), jnp.float32),
        ],
    )(ce_r, gl_r)

    n = jnp.sum(npos)
    return (jnp.sum(loc_part) / n, jnp.sum(cls_part) / n)
